# initial kernel scaffold (unmeasured)
import jax
import jax.numpy as jnp
from jax import lax
from jax.experimental import pallas as pl
from jax.experimental.pallas import tpu as pltpu

N_DEV = 4
B, H, D, BS = 16, 16, 64, 16
NB = 128
P = 128
KEYS = P * BS
SCALE = D ** -0.5
NEG = -1e30


def _partial_body(q_ref, k_ref, v_ref, bt_ref, lens_ref, out_ref, logw_ref):
    h = pl.program_id(0)

    @pl.when(h == 0)
    def _():
        my_lo = lax.axis_index("i") * P
        bt3 = bt_ref[...][:, :, None]
        p_iota = lax.broadcasted_iota(jnp.int32, (B, NB, P), 2)
        j_iota = lax.broadcasted_iota(jnp.int32, (B, NB, P), 1)
        lens3 = lens_ref[...][:, :, None]
        hit = (bt3 == p_iota + my_lo) & (j_iota < lens3)
        counts = jnp.sum(hit.astype(jnp.float32), axis=1)
        kp = lax.broadcasted_iota(jnp.int32, (P, KEYS), 1) // BS
        pp = lax.broadcasted_iota(jnp.int32, (P, KEYS), 0)
        expand = (kp == pp).astype(jnp.float32)
        w = lax.dot(counts, expand, preferred_element_type=jnp.float32)
        logw_ref[...] = jnp.where(w > 0, jnp.log(w), NEG)

    q = q_ref[...].reshape(B, D).astype(jnp.bfloat16)
    k = k_ref[...].reshape(KEYS, D).astype(jnp.bfloat16)
    v = v_ref[...].reshape(KEYS, D).astype(jnp.bfloat16)
    s = lax.dot_general(q, k, (((1,), (1,)), ((), ())),
                        preferred_element_type=jnp.float32) * SCALE
    s = s + logw_ref[...]
    m = jnp.max(s, axis=1, keepdims=True)
    e = jnp.exp(s - m)
    l = jnp.sum(e, axis=1, keepdims=True)
    o = lax.dot(e.astype(jnp.bfloat16), v,
                preferred_element_type=jnp.float32)
    out_ref[:, 0, 0:D] = o
    out_ref[:, 0, D:D + 1] = m
    out_ref[:, 0, D + 1:D + 2] = l


def _combine_body(p_ref, out_ref, comm_ref, send_sems, recv_sems):
    my = lax.axis_index("i")
    left = lax.rem(my + N_DEV - 1, N_DEV)
    right = lax.rem(my + 1, N_DEV)

    barrier = pltpu.get_barrier_semaphore()
    for nbr in (left, right):
        pl.semaphore_signal(barrier, inc=1, device_id=(nbr,),
                            device_id_type=pl.DeviceIdType.MESH)
    pl.semaphore_wait(barrier, 2)

    comm_ref[0, ...] = p_ref[...]
    for hop in range(N_DEV - 1):
        rdma = pltpu.make_async_remote_copy(
            src_ref=comm_ref.at[hop],
            dst_ref=comm_ref.at[hop + 1],
            send_sem=send_sems.at[hop],
            recv_sem=recv_sems.at[hop],
            device_id=(right,),
            device_id_type=pl.DeviceIdType.MESH,
        )
        rdma.start()
        rdma.wait()

    ms = [comm_ref[k, :, :, D:D + 1] for k in range(N_DEV)]
    gmax = ms[0]
    for k in range(1, N_DEV):
        gmax = jnp.maximum(gmax, ms[k])
    num = jnp.zeros((B, H, D), jnp.float32)
    den = jnp.zeros((B, H, 1), jnp.float32)
    for k in range(N_DEV):
        c = jnp.exp(ms[k] - gmax)
        num = num + c * comm_ref[k, :, :, 0:D]
        den = den + c * comm_ref[k, :, :, D + 1:D + 2]
    out_ref[...] = (num / den)[:, None, :, :]


def kernel(Q, K, V, bt, lens):
    lens2 = lens.reshape(B, 1)

    partial = pl.pallas_call(
        _partial_body,
        grid=(H,),
        in_specs=[
            pl.BlockSpec((B, 1, 1, D), lambda h: (0, 0, h, 0)),
            pl.BlockSpec((P, BS, 1, D), lambda h: (0, 0, h, 0)),
            pl.BlockSpec((P, BS, 1, D), lambda h: (0, 0, h, 0)),
            pl.BlockSpec((B, NB), lambda h: (0, 0)),
            pl.BlockSpec((B, 1), lambda h: (0, 0)),
        ],
        out_specs=pl.BlockSpec((B, 1, 128), lambda h: (0, h, 0)),
        out_shape=jax.ShapeDtypeStruct((B, H, 128), jnp.float32),
        scratch_shapes=[pltpu.VMEM((B, KEYS), jnp.float32)],
    )(Q, K, V, bt, lens2)

    return pl.pallas_call(
        _combine_body,
        out_shape=jax.ShapeDtypeStruct((B, 1, H, D), jnp.float32),
        in_specs=[pl.BlockSpec(memory_space=pltpu.VMEM)],
        out_specs=pl.BlockSpec(memory_space=pltpu.VMEM),
        scratch_shapes=[
            pltpu.VMEM((N_DEV, B, H, 128), jnp.float32),
            pltpu.SemaphoreType.DMA((N_DEV - 1,)),
            pltpu.SemaphoreType.DMA((N_DEV - 1,)),
        ],
        compiler_params=pltpu.CompilerParams(collective_id=0),
    )(partial)


# baseline (device time: 73161 ns/iter reference)
import jax
import jax.numpy as jnp
from jax import lax
from jax.experimental import pallas as pl
from jax.experimental.pallas import tpu as pltpu

N_DEV = 4
B, H, D, BS = 16, 16, 64, 16
NB = 128
P = 128
KEYS = P * BS
HPB = 2
PK = 128
SCALE = D ** -0.5
NEG = -1e30


def _partial_body(q_ref, k_ref, v_ref, bt_ref, lens_ref, out_ref, logw_ref):
    g = pl.program_id(0)

    @pl.when(g == 0)
    def _():
        my_lo = lax.axis_index("i") * P
        bt3 = bt_ref[...][:, :, None]
        p_iota = lax.broadcasted_iota(jnp.int32, (B, NB, P), 2)
        j_iota = lax.broadcasted_iota(jnp.int32, (B, NB, P), 1)
        lens3 = lens_ref[...][:, :, None]
        hit = (bt3 == p_iota + my_lo) & (j_iota < lens3)
        counts = jnp.sum(hit.astype(jnp.float32), axis=1)
        kp = lax.broadcasted_iota(jnp.int32, (P, KEYS), 1) // BS
        pp = lax.broadcasted_iota(jnp.int32, (P, KEYS), 0)
        expand = (kp == pp).astype(jnp.float32)
        w = lax.dot(counts, expand, preferred_element_type=jnp.float32)
        logw_ref[...] = jnp.where(w > 0, jnp.log(w), NEG)

    logw = logw_ref[...]
    for sub in range(HPB):
        q = q_ref[:, sub * D:(sub + 1) * D].astype(jnp.bfloat16)
        k = k_ref[:, sub * D:(sub + 1) * D].astype(jnp.bfloat16)
        v = v_ref[:, sub * D:(sub + 1) * D].astype(jnp.bfloat16)
        s = lax.dot_general(q, k, (((1,), (1,)), ((), ())),
                            preferred_element_type=jnp.float32) * SCALE
        s = s + logw
        m = jnp.max(s, axis=1, keepdims=True)
        e = jnp.exp(s - m)
        l = jnp.sum(e, axis=1, keepdims=True)
        o = lax.dot(e.astype(jnp.bfloat16), v,
                    preferred_element_type=jnp.float32)
        base = sub * PK
        out_ref[:, base:base + D] = o
        out_ref[:, base + D:base + D + 1] = m
        out_ref[:, base + D + 1:base + D + 2] = l


def _combine_body(p_ref, out_ref, comm_ref, send_sems, recv_sems):
    my = lax.axis_index("i")
    left = lax.rem(my + N_DEV - 1, N_DEV)
    right = lax.rem(my + 1, N_DEV)

    barrier = pltpu.get_barrier_semaphore()
    for nbr in (left, right):
        pl.semaphore_signal(barrier, inc=1, device_id=(nbr,),
                            device_id_type=pl.DeviceIdType.MESH)
    pl.semaphore_wait(barrier, 2)

    comm_ref[0, ...] = p_ref[...]
    for hop in range(N_DEV - 1):
        rdma = pltpu.make_async_remote_copy(
            src_ref=comm_ref.at[hop],
            dst_ref=comm_ref.at[hop + 1],
            send_sem=send_sems.at[hop],
            recv_sem=recv_sems.at[hop],
            device_id=(right,),
            device_id_type=pl.DeviceIdType.MESH,
        )
        rdma.start()
        rdma.wait()

    ms = [comm_ref[k, :, :, D:D + 1] for k in range(N_DEV)]
    gmax = ms[0]
    for k in range(1, N_DEV):
        gmax = jnp.maximum(gmax, ms[k])
    num = jnp.zeros((B, H, D), jnp.float32)
    den = jnp.zeros((B, H, 1), jnp.float32)
    for k in range(N_DEV):
        c = jnp.exp(ms[k] - gmax)
        num = num + c * comm_ref[k, :, :, 0:D]
        den = den + c * comm_ref[k, :, :, D + 1:D + 2]
    out_ref[...] = (num / den)[:, None, :, :]


def kernel(Q, K, V, bt, lens):
    q2 = Q.reshape(B, H * D)
    k2 = K.reshape(KEYS, H * D)
    v2 = V.reshape(KEYS, H * D)
    lens2 = lens.reshape(B, 1)

    partial = pl.pallas_call(
        _partial_body,
        grid=(H // HPB,),
        in_specs=[
            pl.BlockSpec((B, HPB * D), lambda g: (0, g)),
            pl.BlockSpec((KEYS, HPB * D), lambda g: (0, g)),
            pl.BlockSpec((KEYS, HPB * D), lambda g: (0, g)),
            pl.BlockSpec((B, NB), lambda g: (0, 0)),
            pl.BlockSpec((B, 1), lambda g: (0, 0)),
        ],
        out_specs=pl.BlockSpec((B, HPB * PK), lambda g: (0, g)),
        out_shape=jax.ShapeDtypeStruct((B, H * PK), jnp.float32),
        scratch_shapes=[pltpu.VMEM((B, KEYS), jnp.float32)],
    )(q2, k2, v2, bt, lens2)

    return pl.pallas_call(
        _combine_body,
        out_shape=jax.ShapeDtypeStruct((B, 1, H, D), jnp.float32),
        in_specs=[pl.BlockSpec(memory_space=pltpu.VMEM)],
        out_specs=pl.BlockSpec(memory_space=pltpu.VMEM),
        scratch_shapes=[
            pltpu.VMEM((N_DEV, B, H, PK), jnp.float32),
            pltpu.SemaphoreType.DMA((N_DEV - 1,)),
            pltpu.SemaphoreType.DMA((N_DEV - 1,)),
        ],
        compiler_params=pltpu.CompilerParams(collective_id=0),
    )(partial.reshape(B, H, PK))


# device time: 28851 ns/iter; 2.5358x vs baseline; 2.5358x over previous
import jax
import jax.numpy as jnp
from jax import lax
from jax.experimental import pallas as pl
from jax.experimental.pallas import tpu as pltpu

N_DEV = 4
B, H, D, BS = 16, 16, 64, 16
NB = 128
P = 128
KEYS = P * BS
HD = H * D
R = H * B
PK = 128
SCALE = D ** -0.5
NEG = -1e30


def _body(q_ref, k_ref, v_ref, bt_ref, lens_ref, out_ref,
          comm_ref, send_sems, recv_sems):
    my = lax.axis_index("i")
    left = lax.rem(my + N_DEV - 1, N_DEV)
    right = lax.rem(my + 1, N_DEV)
    barrier = pltpu.get_barrier_semaphore()
    for nbr in (left, right):
        pl.semaphore_signal(barrier, inc=1, device_id=(nbr,),
                            device_id_type=pl.DeviceIdType.MESH)
    pl.semaphore_wait(barrier, 2)

    my_lo = my * P
    bt3 = bt_ref[...][:, :, None]
    p_iota = lax.broadcasted_iota(jnp.int32, (B, NB, P), 2)
    j_iota = lax.broadcasted_iota(jnp.int32, (B, NB, P), 1)
    lens3 = lens_ref[...][:, :, None]
    hit = (bt3 == p_iota + my_lo) & (j_iota < lens3)
    counts = jnp.sum(hit.astype(jnp.float32), axis=1)
    kp = lax.broadcasted_iota(jnp.int32, (P, KEYS), 1) % P
    pp = lax.broadcasted_iota(jnp.int32, (P, KEYS), 0)
    expand = (kp == pp).astype(jnp.float32)
    w = lax.dot(counts, expand, preferred_element_type=jnp.float32)
    logw = jnp.where(w > 0, jnp.log(w), NEG)

    q2 = q_ref[...].reshape(B, HD)
    lane_h = lax.broadcasted_iota(jnp.int32, (H, B, HD), 2) // D
    h_idx = lax.broadcasted_iota(jnp.int32, (H, B, HD), 0)
    sel3 = (lane_h == h_idx)
    qm = jnp.where(sel3, q2[None, :, :], 0.0)
    qm = qm.reshape(R, HD).astype(jnp.bfloat16)

    kb = k_ref[...].astype(jnp.bfloat16).reshape(BS, HD, P)
    vb = v_ref[...].astype(jnp.bfloat16).reshape(BS, HD, P)

    s = jnp.concatenate(
        [lax.dot(qm, kb[sl], preferred_element_type=jnp.float32)
         for sl in range(BS)], axis=1) * SCALE
    s = s + jnp.broadcast_to(logw[None], (H, B, KEYS)).reshape(R, KEYS)
    m = jnp.max(s, axis=1, keepdims=True)
    e = jnp.exp(s - m)
    l = jnp.sum(e, axis=1, keepdims=True)
    eb = e.astype(jnp.bfloat16)

    o_full = jnp.zeros((R, HD), jnp.float32)
    for sl in range(BS):
        o_full = o_full + lax.dot_general(
            eb[:, sl * P:(sl + 1) * P], vb[sl],
            (((1,), (1,)), ((), ())),
            preferred_element_type=jnp.float32)
    o_masked = o_full * sel3.reshape(R, HD).astype(jnp.float32)
    g_iota = lax.broadcasted_iota(jnp.int32, (HD, D), 0)
    d_iota = lax.broadcasted_iota(jnp.int32, (HD, D), 1)
    fold = (g_iota % D == d_iota).astype(jnp.float32)
    o_sel = lax.dot(o_masked, fold,
                    preferred_element_type=jnp.float32)

    for h in range(H):
        rows = slice(h * B, (h + 1) * B)
        comm_ref[0, :, h, 0:D] = o_sel[rows, :]
        comm_ref[0, :, h, D:D + 1] = m[rows, :]
        comm_ref[0, :, h, D + 1:D + 2] = l[rows, :]

    for hop in range(N_DEV - 1):
        rdma = pltpu.make_async_remote_copy(
            src_ref=comm_ref.at[hop],
            dst_ref=comm_ref.at[hop + 1],
            send_sem=send_sems.at[hop],
            recv_sem=recv_sems.at[hop],
            device_id=(right,),
            device_id_type=pl.DeviceIdType.MESH,
        )
        rdma.start()
        rdma.wait()

    ms = [comm_ref[k, :, :, D:D + 1] for k in range(N_DEV)]
    gmax = ms[0]
    for k in range(1, N_DEV):
        gmax = jnp.maximum(gmax, ms[k])
    num = jnp.zeros((B, H, D), jnp.float32)
    den = jnp.zeros((B, H, 1), jnp.float32)
    for k in range(N_DEV):
        c = jnp.exp(ms[k] - gmax)
        num = num + c * comm_ref[k, :, :, 0:D]
        den = den + c * comm_ref[k, :, :, D + 1:D + 2]
    out_ref[...] = (num / den)[:, None, :, :]


def kernel(Q, K, V, bt, lens):
    kt = jnp.transpose(K, (1, 2, 3, 0))
    vt = jnp.transpose(V, (1, 2, 3, 0))
    return pl.pallas_call(
        _body,
        out_shape=jax.ShapeDtypeStruct((B, 1, H, D), jnp.float32),
        in_specs=[pl.BlockSpec(memory_space=pltpu.VMEM)] * 5,
        out_specs=pl.BlockSpec(memory_space=pltpu.VMEM),
        scratch_shapes=[
            pltpu.VMEM((N_DEV, B, H, PK), jnp.float32),
            pltpu.SemaphoreType.DMA((N_DEV - 1,)),
            pltpu.SemaphoreType.DMA((N_DEV - 1,)),
        ],
        compiler_params=pltpu.CompilerParams(collective_id=0),
    )(Q, kt, vt, bt, lens.reshape(B, 1))


# device time: 19495 ns/iter; 3.7528x vs baseline; 1.4799x over previous
import jax
import jax.numpy as jnp
from jax import lax
from jax.experimental import pallas as pl
from jax.experimental.pallas import tpu as pltpu

N_DEV = 4
B, H, D, BS = 16, 16, 64, 16
NB = 128
P = 128
KEYS = P * BS
HD = H * D
R = H * B
PK = 128
SCALE = D ** -0.5
NEG = -1e30


def _body(q_ref, k_ref, v_ref, bt_ref, lens_ref, out_ref,
          k_vmem, v_vmem, comm_ref, load_sems, send_sems, recv_sems):
    k_dma = pltpu.make_async_copy(k_ref, k_vmem, load_sems.at[0])
    v_dma = pltpu.make_async_copy(v_ref, v_vmem, load_sems.at[1])
    k_dma.start()
    v_dma.start()

    my = lax.axis_index("i")
    barrier = pltpu.get_barrier_semaphore()
    for off in (1, 2, 3):
        pl.semaphore_signal(barrier, inc=1,
                            device_id=(lax.rem(my + off, N_DEV),),
                            device_id_type=pl.DeviceIdType.MESH)
    pl.semaphore_wait(barrier, 3)

    my_lo = my * P
    bt3 = bt_ref[...][:, :, None]
    p_iota = lax.broadcasted_iota(jnp.int32, (B, NB, P), 2)
    j_iota = lax.broadcasted_iota(jnp.int32, (B, NB, P), 1)
    lens3 = lens_ref[...][:, :, None]
    hit = (bt3 == p_iota + my_lo) & (j_iota < lens3)
    counts = jnp.sum(hit.astype(jnp.float32), axis=1)
    kp = lax.broadcasted_iota(jnp.int32, (P, KEYS), 1) % P
    pp = lax.broadcasted_iota(jnp.int32, (P, KEYS), 0)
    expand = (kp == pp).astype(jnp.float32)
    w = lax.dot(counts, expand, preferred_element_type=jnp.float32)
    logw = jnp.where(w > 0, jnp.log(w), NEG)

    q2 = q_ref[...].reshape(B, HD)
    lane_h = lax.broadcasted_iota(jnp.int32, (H, B, HD), 2) // D
    h_idx = lax.broadcasted_iota(jnp.int32, (H, B, HD), 0)
    sel3 = (lane_h == h_idx)
    qm = jnp.where(sel3, q2[None, :, :], 0.0)
    qm = qm.reshape(R, HD).astype(jnp.bfloat16)

    k_dma.wait()
    kb = k_vmem[...].astype(jnp.bfloat16).reshape(BS, HD, P)

    s = jnp.concatenate(
        [lax.dot(qm, kb[sl], preferred_element_type=jnp.float32)
         for sl in range(BS)], axis=1) * SCALE
    s = s + jnp.broadcast_to(logw[None], (H, B, KEYS)).reshape(R, KEYS)
    m = jnp.max(s, axis=1, keepdims=True)
    e = jnp.exp(s - m)
    l = jnp.sum(e, axis=1, keepdims=True)
    eb = e.astype(jnp.bfloat16)

    v_dma.wait()
    vb = v_vmem[...].astype(jnp.bfloat16).reshape(BS, HD, P)

    o_full = jnp.zeros((R, HD), jnp.float32)
    for sl in range(BS):
        o_full = o_full + lax.dot_general(
            eb[:, sl * P:(sl + 1) * P], vb[sl],
            (((1,), (1,)), ((), ())),
            preferred_element_type=jnp.float32)
    o_masked = o_full * sel3.reshape(R, HD).astype(jnp.float32)
    g_iota = lax.broadcasted_iota(jnp.int32, (HD, D), 0)
    d_iota = lax.broadcasted_iota(jnp.int32, (HD, D), 1)
    fold = (g_iota % D == d_iota).astype(jnp.float32)
    o_sel = lax.dot(o_masked, fold,
                    preferred_element_type=jnp.float32)

    for h in range(H):
        rows = slice(h * B, (h + 1) * B)
        comm_ref[0, :, h, 0:D] = o_sel[rows, :]
        comm_ref[0, :, h, D:D + 1] = m[rows, :]
        comm_ref[0, :, h, D + 1:D + 2] = l[rows, :]

    rdmas = []
    for off in (1, 2, 3):
        slot = N_DEV - off
        rdma = pltpu.make_async_remote_copy(
            src_ref=comm_ref.at[0],
            dst_ref=comm_ref.at[slot],
            send_sem=send_sems.at[slot],
            recv_sem=recv_sems.at[slot],
            device_id=(lax.rem(my + off, N_DEV),),
            device_id_type=pl.DeviceIdType.MESH,
        )
        rdma.start()
        rdmas.append(rdma)
    for rdma in rdmas:
        rdma.wait()

    ms = [comm_ref[k, :, :, D:D + 1] for k in range(N_DEV)]
    gmax = ms[0]
    for k in range(1, N_DEV):
        gmax = jnp.maximum(gmax, ms[k])
    num = jnp.zeros((B, H, D), jnp.float32)
    den = jnp.zeros((B, H, 1), jnp.float32)
    for k in range(N_DEV):
        c = jnp.exp(ms[k] - gmax)
        num = num + c * comm_ref[k, :, :, 0:D]
        den = den + c * comm_ref[k, :, :, D + 1:D + 2]
    out_ref[...] = (num / den)[:, None, :, :]


def kernel(Q, K, V, bt, lens):
    kt = jnp.transpose(K, (1, 2, 3, 0))
    vt = jnp.transpose(V, (1, 2, 3, 0))
    hbm = pltpu.MemorySpace.HBM
    kt = pltpu.with_memory_space_constraint(kt, hbm)
    vt = pltpu.with_memory_space_constraint(vt, hbm)
    return pl.pallas_call(
        _body,
        out_shape=jax.ShapeDtypeStruct((B, 1, H, D), jnp.float32),
        in_specs=[
            pl.BlockSpec(memory_space=pltpu.MemorySpace.VMEM),
            pl.BlockSpec(memory_space=hbm),
            pl.BlockSpec(memory_space=hbm),
            pl.BlockSpec(memory_space=pltpu.MemorySpace.VMEM),
            pl.BlockSpec(memory_space=pltpu.MemorySpace.VMEM),
        ],
        out_specs=pl.BlockSpec(memory_space=pltpu.MemorySpace.VMEM),
        scratch_shapes=[
            pltpu.VMEM((BS, H, D, P), jnp.float32),
            pltpu.VMEM((BS, H, D, P), jnp.float32),
            pltpu.VMEM((N_DEV, B, H, PK), jnp.float32),
            pltpu.SemaphoreType.DMA((2,)),
            pltpu.SemaphoreType.DMA((N_DEV,)),
            pltpu.SemaphoreType.DMA((N_DEV,)),
        ],
        compiler_params=pltpu.CompilerParams(collective_id=0),
    )(Q, kt, vt, bt, lens.reshape(B, 1))
